# baseline (speedup 1.0000x reference)
import functools

import jax
import jax.numpy as jnp
from jax import lax
from jax.experimental import pallas as pl
from jax.experimental.pallas import tpu as pltpu
from jax.experimental.pallas import tpu_sc as plsc

_NC = 2
_NS = 16
_NW = _NC * _NS
_CHUNK = 128
_BNC = float(1.0 / (1.0 + 1e-5) ** 0.5)


def _vsc_mesh():
    return plsc.VectorSubcoreMesh(
        core_axis_name="c", subcore_axis_name="s",
        num_cores=_NC, num_subcores=_NS)


def _sc_degree(col_r, n_pad):
    K = col_r.shape[1]
    rows_pt = n_pad // _NS

    @functools.partial(
        pl.kernel,
        out_type=jax.ShapeDtypeStruct((_NC, n_pad, 16), jnp.float32),
        mesh=_vsc_mesh(),
        scratch_types=[
            pltpu.VMEM((K, _CHUNK), jnp.int32),
            pltpu.VMEM((_CHUNK, 16), jnp.float32),
            pltpu.VMEM((_CHUNK, 16), jnp.float32),
            pltpu.VMEM_SHARED((n_pad, 16), jnp.float32),
        ],
    )
    def deg_kernel(col_h, out_h, cidx, ones_v, stage_v, acc):
        c = lax.axis_index("c")
        s = lax.axis_index("s")
        w = s * _NC + c
        one16 = jnp.full((16,), 1.0, jnp.float32)
        zero16 = jnp.zeros((16,), jnp.float32)

        def fill(r, carry):
            ones_v[r, :] = one16
            stage_v[r, :] = zero16
            return carry
        lax.fori_loop(0, _CHUNK, fill, 0)

        base = s * rows_pt
        for i in range(rows_pt // _CHUNK):
            pltpu.sync_copy(stage_v, acc.at[pl.ds(base + i * _CHUNK, _CHUNK)])
        plsc.subcore_barrier()

        pltpu.sync_copy(col_h.at[w], cidx)

        def body(j, carry):
            pltpu.sync_copy(ones_v, acc.at[cidx.at[j]], add=True)
            return carry
        lax.fori_loop(0, K, body, 0)
        plsc.subcore_barrier()

        for i in range(rows_pt // _CHUNK):
            sl = pl.ds(base + i * _CHUNK, _CHUNK)
            pltpu.sync_copy(acc.at[sl], stage_v)
            pltpu.sync_copy(stage_v, out_h.at[c, sl])

    return deg_kernel(col_r)


def _sc_aggregate(table, row_r, col_r, n_pad):
    K = row_r.shape[1]
    Hk = table.shape[1]
    rows_pt = n_pad // _NS

    @functools.partial(
        pl.kernel,
        out_type=jax.ShapeDtypeStruct((_NC, n_pad, Hk), jnp.float32),
        mesh=_vsc_mesh(),
        compiler_params=pltpu.CompilerParams(use_tc_tiling_on_sc=False),
        scratch_types=[
            pltpu.VMEM((K, _CHUNK), jnp.int32),
            pltpu.VMEM((K, _CHUNK), jnp.int32),
            pltpu.VMEM((_CHUNK, Hk), jnp.float32),
            pltpu.VMEM((_CHUNK, Hk), jnp.float32),
            pltpu.VMEM_SHARED((n_pad, Hk), jnp.float32),
            pltpu.SemaphoreType.DMA,
            pltpu.SemaphoreType.DMA,
        ],
    )
    def agg_kernel(table_h, row_h, col_h, out_h, ridx, cidx, gb0, gb1, acc,
                   sem0, sem1):
        c = lax.axis_index("c")
        s = lax.axis_index("s")
        w = s * _NC + c
        zero16 = jnp.zeros((16,), jnp.float32)

        def fill(r, carry):
            for jj in range(Hk // 16):
                gb0[r, pl.ds(jj * 16, 16)] = zero16
            return carry
        lax.fori_loop(0, _CHUNK, fill, 0)

        base = s * rows_pt
        for i in range(rows_pt // _CHUNK):
            pltpu.sync_copy(gb0, acc.at[pl.ds(base + i * _CHUNK, _CHUNK)])
        plsc.subcore_barrier()

        pltpu.sync_copy(row_h.at[w], ridx)
        pltpu.sync_copy(col_h.at[w], cidx)

        def body(j2, carry):
            j = 2 * j2
            d0 = pltpu.async_copy(table_h.at[ridx.at[j]], gb0, sem0)
            d1 = pltpu.async_copy(table_h.at[ridx.at[j + 1]], gb1, sem1)
            d0.wait()
            pltpu.sync_copy(gb0, acc.at[cidx.at[j]], add=True)
            d1.wait()
            pltpu.sync_copy(gb1, acc.at[cidx.at[j + 1]], add=True)
            return carry
        lax.fori_loop(0, K // 2, body, 0)
        plsc.subcore_barrier()

        for i in range(rows_pt // _CHUNK):
            sl = pl.ds(base + i * _CHUNK, _CHUNK)
            pltpu.sync_copy(acc.at[sl], gb0)
            pltpu.sync_copy(gb0, out_h.at[c, sl])

    return agg_kernel(table, row_r, col_r)


def _tc_matmul(x, W, br):
    n, d = x.shape
    h = W.shape[1]

    def body(xr, wr, orf):
        orf[...] = jnp.dot(xr[...], wr[...], preferred_element_type=jnp.float32)

    return pl.pallas_call(
        body,
        grid=(n // br,),
        in_specs=[pl.BlockSpec((br, d), lambda i: (i, 0)),
                  pl.BlockSpec((d, h), lambda i: (0, 0))],
        out_specs=pl.BlockSpec((br, h), lambda i: (i, 0)),
        out_shape=jax.ShapeDtypeStruct((n, h), jnp.float32),
    )(x, W)


def _tc_scale(deg2, h1, br):
    n, h = h1.shape

    def body(dr, hr, hs_ref, dv_ref):
        d = dr[...]
        deg = d[0, :, 0:1] + d[1, :, 0:1] + 1.0
        dinv = lax.rsqrt(deg)
        hs_ref[...] = dinv * hr[...]
        dv_ref[...] = jnp.broadcast_to(dinv, (br, 8))

    return pl.pallas_call(
        body,
        grid=(n // br,),
        in_specs=[pl.BlockSpec((2, br, 16), lambda i: (0, i, 0)),
                  pl.BlockSpec((br, h), lambda i: (i, 0))],
        out_specs=[pl.BlockSpec((br, h), lambda i: (i, 0)),
                   pl.BlockSpec((br, 8), lambda i: (i, 0))],
        out_shape=[jax.ShapeDtypeStruct((n, h), jnp.float32),
                   jax.ShapeDtypeStruct((n, 8), jnp.float32)],
    )(deg2, h1)


def _tc_fuse(agg, hs, dinv8, b, g, be, W, br, rescale, out_bias=None):
    n, h = hs.shape
    h_out = W.shape[1]
    has_ob = out_bias is not None

    def body(*refs):
        if has_ob:
            ar, hr, dvr, br_, gr, ber, wr, obr, orf = refs
        else:
            ar, hr, dvr, br_, gr, ber, wr, orf = refs
        a = ar[...]
        dv = dvr[...][:, 0:1]
        t = dv * (a[0] + a[1] + hr[...]) + br_[...]
        t = jnp.maximum((_BNC * gr[...]) * t + ber[...], 0.0)
        o = jnp.dot(t, wr[...], preferred_element_type=jnp.float32)
        if has_ob:
            o = o + obr[...]
        if rescale:
            o = dv * o
        orf[...] = o

    in_specs = [pl.BlockSpec((2, br, h), lambda i: (0, i, 0)),
                pl.BlockSpec((br, h), lambda i: (i, 0)),
                pl.BlockSpec((br, 8), lambda i: (i, 0)),
                pl.BlockSpec((h,), lambda i: (0,)),
                pl.BlockSpec((h,), lambda i: (0,)),
                pl.BlockSpec((h,), lambda i: (0,)),
                pl.BlockSpec((h, h_out), lambda i: (0, 0))]
    args = [agg, hs, dinv8, b, g, be, W]
    if has_ob:
        in_specs.append(pl.BlockSpec((h_out,), lambda i: (0,)))
        args.append(out_bias)
    return pl.pallas_call(
        body,
        grid=(n // br,),
        in_specs=in_specs,
        out_specs=pl.BlockSpec((br, h_out), lambda i: (i, 0)),
        out_shape=jax.ShapeDtypeStruct((n, h_out), jnp.float32),
    )(*args)


def kernel(x, edge_index, W1, b1, g1, be1, W2, b2, g2, be2, Wl, bl):
    n, d = x.shape
    e = edge_index.shape[1]
    out_w = Wl.shape[1]

    k_chunks = -(-e // (_NW * _CHUNK))
    if k_chunks % 2:
        k_chunks += 1
    e_pad = _NW * k_chunks * _CHUNK
    n_pad = -(-(n + 1) // (_NS * _CHUNK)) * (_NS * _CHUNK)

    row = edge_index[0]
    col = edge_index[1]
    row_r = jnp.concatenate(
        [row, jnp.zeros((e_pad - e,), jnp.int32)]).reshape(_NW, k_chunks, _CHUNK)
    col_r = jnp.concatenate(
        [col, jnp.full((e_pad - e,), n, jnp.int32)]).reshape(_NW, k_chunks, _CHUNK)

    br = 1000 if n % 1000 == 0 else 8 * (n // 8)

    deg2 = _sc_degree(col_r, n_pad)
    h1 = _tc_matmul(x, W1, br)
    hs1, dinv8 = _tc_scale(deg2, h1, br)

    agg1 = _sc_aggregate(hs1, row_r, col_r, n_pad)
    hs2 = _tc_fuse(agg1, hs1, dinv8, b1, g1, be1, W2, br, rescale=True)

    agg2 = _sc_aggregate(hs2, row_r, col_r, n_pad)
    wl_pad = jnp.zeros((Wl.shape[0], 128), jnp.float32).at[:, :out_w].set(Wl)
    bl_pad = jnp.zeros((128,), jnp.float32).at[:out_w].set(bl)
    out = _tc_fuse(agg2, hs2, dinv8, b2, g2, be2, wl_pad, br, rescale=False,
                   out_bias=bl_pad)
    return out[:, :out_w]


# speedup vs baseline: 1.9263x; 1.9263x over previous
import functools

import jax
import jax.numpy as jnp
from jax import lax
from jax.experimental import pallas as pl
from jax.experimental.pallas import tpu as pltpu
from jax.experimental.pallas import tpu_sc as plsc

_NC = 2
_NS = 16
_NW = _NC * _NS
_CHUNK = 128
_BNC = float(1.0 / (1.0 + 1e-5) ** 0.5)


def _vsc_mesh():
    return plsc.VectorSubcoreMesh(
        core_axis_name="c", subcore_axis_name="s",
        num_cores=_NC, num_subcores=_NS)


def _sc_degree(col_r, n_pad):
    K = col_r.shape[1]
    rows_pt = n_pad // _NS

    @functools.partial(
        pl.kernel,
        out_type=jax.ShapeDtypeStruct((_NC, n_pad, 16), jnp.float32),
        mesh=_vsc_mesh(),
        scratch_types=[
            pltpu.VMEM((K, _CHUNK), jnp.int32),
            pltpu.VMEM((_CHUNK, 16), jnp.float32),
            pltpu.VMEM((_CHUNK, 16), jnp.float32),
            pltpu.VMEM_SHARED((n_pad, 16), jnp.float32),
        ],
    )
    def deg_kernel(col_h, out_h, cidx, ones_v, stage_v, acc):
        c = lax.axis_index("c")
        s = lax.axis_index("s")
        w = s * _NC + c
        one16 = jnp.full((16,), 1.0, jnp.float32)
        zero16 = jnp.zeros((16,), jnp.float32)

        def fill(r, carry):
            ones_v[r, :] = one16
            stage_v[r, :] = zero16
            return carry
        lax.fori_loop(0, _CHUNK, fill, 0)

        base = s * rows_pt
        for i in range(rows_pt // _CHUNK):
            pltpu.sync_copy(stage_v, acc.at[pl.ds(base + i * _CHUNK, _CHUNK)])
        plsc.subcore_barrier()

        pltpu.sync_copy(col_h.at[w], cidx)

        def body(j, carry):
            pltpu.sync_copy(ones_v, acc.at[cidx.at[j]], add=True)
            return carry
        lax.fori_loop(0, K, body, 0)
        plsc.subcore_barrier()

        for i in range(rows_pt // _CHUNK):
            sl = pl.ds(base + i * _CHUNK, _CHUNK)
            pltpu.sync_copy(acc.at[sl], stage_v)
            pltpu.sync_copy(stage_v, out_h.at[c, sl])

    return deg_kernel(col_r)


def _sc_aggregate(table, row_r, col_r, n_pad):
    K = row_r.shape[1]
    Hk = table.shape[1]
    rows_pt = n_pad // _NS

    @functools.partial(
        pl.kernel,
        out_type=jax.ShapeDtypeStruct((_NC, n_pad, Hk), jnp.float32),
        mesh=_vsc_mesh(),
        compiler_params=pltpu.CompilerParams(use_tc_tiling_on_sc=False),
        scratch_types=[
            pltpu.VMEM((K, _CHUNK), jnp.int32),
            pltpu.VMEM((K, _CHUNK), jnp.int32),
            pltpu.VMEM((_CHUNK, Hk), jnp.float32),
            pltpu.VMEM_SHARED((n_pad, Hk), jnp.float32),
            pltpu.VMEM_SHARED((n_pad, Hk), jnp.float32),
        ],
    )
    def agg_kernel(table_h, row_h, col_h, out_h, ridx, cidx, gbuf, tab, acc):
        c = lax.axis_index("c")
        s = lax.axis_index("s")
        w = s * _NC + c
        zero16 = jnp.zeros((16,), jnp.float32)

        def fill(r, carry):
            for jj in range(Hk // 16):
                gbuf[r, pl.ds(jj * 16, 16)] = zero16
            return carry
        lax.fori_loop(0, _CHUNK, fill, 0)

        base = s * rows_pt
        sl_mine = pl.ds(base, rows_pt)
        for i in range(rows_pt // _CHUNK):
            pltpu.sync_copy(gbuf, acc.at[pl.ds(base + i * _CHUNK, _CHUNK)])
        pltpu.sync_copy(table_h.at[sl_mine], tab.at[sl_mine])
        plsc.subcore_barrier()

        pltpu.sync_copy(row_h.at[w], ridx)
        pltpu.sync_copy(col_h.at[w], cidx)

        def body(j, carry):
            pltpu.sync_copy(tab.at[ridx.at[j]], gbuf)
            pltpu.sync_copy(gbuf, acc.at[cidx.at[j]], add=True)
            return carry
        lax.fori_loop(0, K, body, 0)
        plsc.subcore_barrier()

        for i in range(rows_pt // _CHUNK):
            sl = pl.ds(base + i * _CHUNK, _CHUNK)
            pltpu.sync_copy(acc.at[sl], gbuf)
            pltpu.sync_copy(gbuf, out_h.at[c, sl])

    return agg_kernel(table, row_r, col_r)


def _tc_matmul(x, W, br):
    n, d = x.shape
    h = W.shape[1]

    def body(xr, wr, orf):
        orf[...] = jnp.dot(xr[...], wr[...], preferred_element_type=jnp.float32)

    return pl.pallas_call(
        body,
        grid=(n // br,),
        in_specs=[pl.BlockSpec((br, d), lambda i: (i, 0)),
                  pl.BlockSpec((d, h), lambda i: (0, 0))],
        out_specs=pl.BlockSpec((br, h), lambda i: (i, 0)),
        out_shape=jax.ShapeDtypeStruct((n, h), jnp.float32),
    )(x, W)


def _tc_scale(deg2, h1, br, n_pad):
    n, h = h1.shape

    def body(dr, hr, hs_ref, dv_ref):
        d = dr[...]
        deg = d[0, :, 0:1] + d[1, :, 0:1] + 1.0
        dinv = lax.rsqrt(deg)
        hs_ref[...] = dinv * hr[...]
        dv_ref[...] = jnp.broadcast_to(dinv, (br, 8))

    return pl.pallas_call(
        body,
        grid=(n // br,),
        in_specs=[pl.BlockSpec((2, br, 16), lambda i: (0, i, 0)),
                  pl.BlockSpec((br, h), lambda i: (i, 0))],
        out_specs=[pl.BlockSpec((br, h), lambda i: (i, 0)),
                   pl.BlockSpec((br, 8), lambda i: (i, 0))],
        out_shape=[jax.ShapeDtypeStruct((n_pad, h), jnp.float32),
                   jax.ShapeDtypeStruct((n, 8), jnp.float32)],
    )(deg2, h1)


def _tc_fuse(agg, hs, dinv8, b, g, be, W, br, rescale, out_bias=None,
             out_rows=None):
    n = dinv8.shape[0]
    h = hs.shape[1]
    h_out = W.shape[1]
    if out_rows is None:
        out_rows = n
    has_ob = out_bias is not None

    def body(*refs):
        if has_ob:
            ar, hr, dvr, br_, gr, ber, wr, obr, orf = refs
        else:
            ar, hr, dvr, br_, gr, ber, wr, orf = refs
        a = ar[...]
        dv = dvr[...][:, 0:1]
        t = dv * (a[0] + a[1] + hr[...]) + br_[...]
        t = jnp.maximum((_BNC * gr[...]) * t + ber[...], 0.0)
        o = jnp.dot(t, wr[...], preferred_element_type=jnp.float32)
        if has_ob:
            o = o + obr[...]
        if rescale:
            o = dv * o
        orf[...] = o

    in_specs = [pl.BlockSpec((2, br, h), lambda i: (0, i, 0)),
                pl.BlockSpec((br, h), lambda i: (i, 0)),
                pl.BlockSpec((br, 8), lambda i: (i, 0)),
                pl.BlockSpec((h,), lambda i: (0,)),
                pl.BlockSpec((h,), lambda i: (0,)),
                pl.BlockSpec((h,), lambda i: (0,)),
                pl.BlockSpec((h, h_out), lambda i: (0, 0))]
    args = [agg, hs, dinv8, b, g, be, W]
    if has_ob:
        in_specs.append(pl.BlockSpec((h_out,), lambda i: (0,)))
        args.append(out_bias)
    return pl.pallas_call(
        body,
        grid=(n // br,),
        in_specs=in_specs,
        out_specs=pl.BlockSpec((br, h_out), lambda i: (i, 0)),
        out_shape=jax.ShapeDtypeStruct((out_rows, h_out), jnp.float32),
    )(*args)


def kernel(x, edge_index, W1, b1, g1, be1, W2, b2, g2, be2, Wl, bl):
    n, d = x.shape
    e = edge_index.shape[1]
    out_w = Wl.shape[1]

    k_chunks = -(-e // (_NW * _CHUNK))
    if k_chunks % 2:
        k_chunks += 1
    e_pad = _NW * k_chunks * _CHUNK
    n_pad = -(-(n + 1) // (_NS * _CHUNK)) * (_NS * _CHUNK)

    row = edge_index[0]
    col = edge_index[1]
    row_r = jnp.concatenate(
        [row, jnp.zeros((e_pad - e,), jnp.int32)]).reshape(_NW, k_chunks, _CHUNK)
    col_r = jnp.concatenate(
        [col, jnp.full((e_pad - e,), n, jnp.int32)]).reshape(_NW, k_chunks, _CHUNK)

    br = 1000 if n % 1000 == 0 else 8 * (n // 8)

    deg2 = _sc_degree(col_r, n_pad)
    h1 = _tc_matmul(x, W1, br)
    hs1, dinv8 = _tc_scale(deg2, h1, br, n_pad)

    agg1 = _sc_aggregate(hs1, row_r, col_r, n_pad)
    hs2 = _tc_fuse(agg1, hs1, dinv8, b1, g1, be1, W2, br, rescale=True,
                   out_rows=n_pad)

    agg2 = _sc_aggregate(hs2, row_r, col_r, n_pad)
    wl_pad = jnp.zeros((Wl.shape[0], 128), jnp.float32).at[:, :out_w].set(Wl)
    bl_pad = jnp.zeros((128,), jnp.float32).at[:out_w].set(bl)
    out = _tc_fuse(agg2, hs2, dinv8, b2, g2, be2, wl_pad, br, rescale=False,
                   out_bias=bl_pad)
    return out[:, :out_w]


# speedup vs baseline: 1.9301x; 1.0020x over previous
import functools

import jax
import jax.numpy as jnp
from jax import lax
from jax.experimental import pallas as pl
from jax.experimental.pallas import tpu as pltpu
from jax.experimental.pallas import tpu_sc as plsc

_NC = 2
_NS = 16
_NW = _NC * _NS
_CHUNK = 128
_BNC = float(1.0 / (1.0 + 1e-5) ** 0.5)


def _vsc_mesh():
    return plsc.VectorSubcoreMesh(
        core_axis_name="c", subcore_axis_name="s",
        num_cores=_NC, num_subcores=_NS)


def _sc_degree(col_r, n_pad):
    K = col_r.shape[1]
    rows_pt = n_pad // _NS

    @functools.partial(
        pl.kernel,
        out_type=jax.ShapeDtypeStruct((_NC, n_pad, 16), jnp.float32),
        mesh=_vsc_mesh(),
        compiler_params=pltpu.CompilerParams(use_tc_tiling_on_sc=False),
        scratch_types=[
            pltpu.VMEM((K, _CHUNK), jnp.int32),
            pltpu.VMEM((_CHUNK, 16), jnp.float32),
            pltpu.VMEM((_CHUNK, 16), jnp.float32),
            pltpu.VMEM_SHARED((n_pad, 16), jnp.float32),
        ],
    )
    def deg_kernel(col_h, out_h, cidx, ones_v, stage_v, acc):
        c = lax.axis_index("c")
        s = lax.axis_index("s")
        w = s * _NC + c
        one16 = jnp.full((16,), 1.0, jnp.float32)
        zero16 = jnp.zeros((16,), jnp.float32)

        def fill(r, carry):
            ones_v[r, :] = one16
            stage_v[r, :] = zero16
            return carry
        lax.fori_loop(0, _CHUNK, fill, 0)

        base = s * rows_pt
        for i in range(rows_pt // _CHUNK):
            pltpu.sync_copy(stage_v, acc.at[pl.ds(base + i * _CHUNK, _CHUNK)])
        plsc.subcore_barrier()

        pltpu.sync_copy(col_h.at[w], cidx)

        def body(j, carry):
            pltpu.sync_copy(ones_v, acc.at[cidx.at[j]], add=True)
            return carry
        lax.fori_loop(0, K, body, 0)
        plsc.subcore_barrier()

        for i in range(rows_pt // _CHUNK):
            sl = pl.ds(base + i * _CHUNK, _CHUNK)
            pltpu.sync_copy(acc.at[sl], stage_v)
            pltpu.sync_copy(stage_v, out_h.at[c, sl])

    return deg_kernel(col_r)


def _sc_aggregate(table, row_r, col_r, n_pad):
    K = row_r.shape[1]
    Hk = table.shape[1]
    rows_pt = n_pad // _NS

    @functools.partial(
        pl.kernel,
        out_type=jax.ShapeDtypeStruct((_NC, n_pad, Hk), jnp.float32),
        mesh=_vsc_mesh(),
        compiler_params=pltpu.CompilerParams(use_tc_tiling_on_sc=False),
        scratch_types=[
            pltpu.VMEM((K, _CHUNK), jnp.int32),
            pltpu.VMEM((K, _CHUNK), jnp.int32),
            pltpu.VMEM((_CHUNK, Hk), jnp.float32),
            pltpu.VMEM_SHARED((n_pad, Hk), jnp.float32),
            pltpu.VMEM_SHARED((n_pad, Hk), jnp.float32),
        ],
    )
    def agg_kernel(table_h, row_h, col_h, out_h, ridx, cidx, gbuf, tab, acc):
        c = lax.axis_index("c")
        s = lax.axis_index("s")
        w = s * _NC + c
        zero16 = jnp.zeros((16,), jnp.float32)

        def fill(r, carry):
            for jj in range(Hk // 16):
                gbuf[r, pl.ds(jj * 16, 16)] = zero16
            return carry
        lax.fori_loop(0, _CHUNK, fill, 0)

        base = s * rows_pt
        sl_mine = pl.ds(base, rows_pt)
        for i in range(rows_pt // _CHUNK):
            pltpu.sync_copy(gbuf, acc.at[pl.ds(base + i * _CHUNK, _CHUNK)])
        pltpu.sync_copy(table_h.at[sl_mine], tab.at[sl_mine])
        plsc.subcore_barrier()

        pltpu.sync_copy(row_h.at[w], ridx)
        pltpu.sync_copy(col_h.at[w], cidx)

        def body(j, carry):
            pltpu.sync_copy(tab.at[ridx.at[j]], gbuf)
            pltpu.sync_copy(gbuf, acc.at[cidx.at[j]], add=True)
            return carry
        lax.fori_loop(0, K, body, 0)
        plsc.subcore_barrier()

        for i in range(rows_pt // _CHUNK):
            sl = pl.ds(base + i * _CHUNK, _CHUNK)
            pltpu.sync_copy(acc.at[sl], gbuf)
            pltpu.sync_copy(gbuf, out_h.at[c, sl])

    return agg_kernel(table, row_r, col_r)


def _tc_matmul(x, W, br):
    n, d = x.shape
    h = W.shape[1]

    def body(xr, wr, orf):
        orf[...] = jnp.dot(xr[...], wr[...], preferred_element_type=jnp.float32)

    return pl.pallas_call(
        body,
        grid=(n // br,),
        in_specs=[pl.BlockSpec((br, d), lambda i: (i, 0)),
                  pl.BlockSpec((d, h), lambda i: (0, 0))],
        out_specs=pl.BlockSpec((br, h), lambda i: (i, 0)),
        out_shape=jax.ShapeDtypeStruct((n, h), jnp.float32),
    )(x, W)


def _tc_scale(deg2, h1, br, n_pad):
    n, h = h1.shape

    def body(dr, hr, hs_ref, dv_ref):
        d = dr[...]
        deg = d[0, :, 0:1] + d[1, :, 0:1] + 1.0
        dinv = lax.rsqrt(deg)
        hs_ref[...] = dinv * hr[...]
        dv_ref[...] = jnp.broadcast_to(dinv, (br, 8))

    return pl.pallas_call(
        body,
        grid=(n // br,),
        in_specs=[pl.BlockSpec((2, br, 16), lambda i: (0, i, 0)),
                  pl.BlockSpec((br, h), lambda i: (i, 0))],
        out_specs=[pl.BlockSpec((br, h), lambda i: (i, 0)),
                   pl.BlockSpec((br, 8), lambda i: (i, 0))],
        out_shape=[jax.ShapeDtypeStruct((n_pad, h), jnp.float32),
                   jax.ShapeDtypeStruct((n, 8), jnp.float32)],
    )(deg2, h1)


def _tc_fuse(agg, hs, dinv8, b, g, be, W, br, rescale, out_bias=None,
             out_rows=None):
    n = dinv8.shape[0]
    h = hs.shape[1]
    h_out = W.shape[1]
    if out_rows is None:
        out_rows = n
    has_ob = out_bias is not None

    def body(*refs):
        if has_ob:
            ar, hr, dvr, br_, gr, ber, wr, obr, orf = refs
        else:
            ar, hr, dvr, br_, gr, ber, wr, orf = refs
        a = ar[...]
        dv = dvr[...][:, 0:1]
        t = dv * (a[0] + a[1] + hr[...]) + br_[...]
        t = jnp.maximum((_BNC * gr[...]) * t + ber[...], 0.0)
        o = jnp.dot(t, wr[...], preferred_element_type=jnp.float32)
        if has_ob:
            o = o + obr[...]
        if rescale:
            o = dv * o
        orf[...] = o

    in_specs = [pl.BlockSpec((2, br, h), lambda i: (0, i, 0)),
                pl.BlockSpec((br, h), lambda i: (i, 0)),
                pl.BlockSpec((br, 8), lambda i: (i, 0)),
                pl.BlockSpec((h,), lambda i: (0,)),
                pl.BlockSpec((h,), lambda i: (0,)),
                pl.BlockSpec((h,), lambda i: (0,)),
                pl.BlockSpec((h, h_out), lambda i: (0, 0))]
    args = [agg, hs, dinv8, b, g, be, W]
    if has_ob:
        in_specs.append(pl.BlockSpec((h_out,), lambda i: (0,)))
        args.append(out_bias)
    return pl.pallas_call(
        body,
        grid=(n // br,),
        in_specs=in_specs,
        out_specs=pl.BlockSpec((br, h_out), lambda i: (i, 0)),
        out_shape=jax.ShapeDtypeStruct((out_rows, h_out), jnp.float32),
    )(*args)


def kernel(x, edge_index, W1, b1, g1, be1, W2, b2, g2, be2, Wl, bl):
    n, d = x.shape
    e = edge_index.shape[1]
    out_w = Wl.shape[1]

    k_chunks = -(-e // (_NW * _CHUNK))
    if k_chunks % 2:
        k_chunks += 1
    e_pad = _NW * k_chunks * _CHUNK
    n_pad = -(-(n + 1) // (_NS * _CHUNK)) * (_NS * _CHUNK)

    row = edge_index[0]
    col = edge_index[1]
    row_r = jnp.concatenate(
        [row, jnp.zeros((e_pad - e,), jnp.int32)]).reshape(_NW, k_chunks, _CHUNK)
    col_r = jnp.concatenate(
        [col, jnp.full((e_pad - e,), n, jnp.int32)]).reshape(_NW, k_chunks, _CHUNK)

    br = 1000 if n % 1000 == 0 else 8 * (n // 8)

    deg2 = _sc_degree(col_r, n_pad)
    h1 = _tc_matmul(x, W1, br)
    hs1, dinv8 = _tc_scale(deg2, h1, br, n_pad)

    agg1 = _sc_aggregate(hs1, row_r, col_r, n_pad)
    hs2 = _tc_fuse(agg1, hs1, dinv8, b1, g1, be1, W2, br, rescale=True,
                   out_rows=n_pad)

    agg2 = _sc_aggregate(hs2, row_r, col_r, n_pad)
    wl_pad = jnp.zeros((Wl.shape[0], 128), jnp.float32).at[:, :out_w].set(Wl)
    bl_pad = jnp.zeros((128,), jnp.float32).at[:out_w].set(bl)
    out = _tc_fuse(agg2, hs2, dinv8, b2, g2, be2, wl_pad, br, rescale=False,
                   out_bias=bl_pad)
    return out[:, :out_w]


# speedup vs baseline: 2.2907x; 1.1868x over previous
import functools

import jax
import jax.numpy as jnp
from jax import lax
from jax.experimental import pallas as pl
from jax.experimental.pallas import tpu as pltpu
from jax.experimental.pallas import tpu_sc as plsc

_NC = 2
_NS = 16
_NW = _NC * _NS
_CHUNK = 128
_BNC = float(1.0 / (1.0 + 1e-5) ** 0.5)


def _vsc_mesh():
    return plsc.VectorSubcoreMesh(
        core_axis_name="c", subcore_axis_name="s",
        num_cores=_NC, num_subcores=_NS)


def _sc_degree(col_r, n_pad):
    K = col_r.shape[1]
    rows_pt = n_pad // _NS

    @functools.partial(
        pl.kernel,
        out_type=jax.ShapeDtypeStruct((_NC, n_pad, 16), jnp.float32),
        mesh=_vsc_mesh(),
        compiler_params=pltpu.CompilerParams(use_tc_tiling_on_sc=False),
        scratch_types=[
            pltpu.VMEM((K, _CHUNK), jnp.int32),
            pltpu.VMEM((_CHUNK, 16), jnp.float32),
            pltpu.VMEM((_CHUNK, 16), jnp.float32),
            pltpu.VMEM_SHARED((n_pad, 16), jnp.float32),
        ],
    )
    def deg_kernel(col_h, out_h, cidx, ones_v, stage_v, acc):
        c = lax.axis_index("c")
        s = lax.axis_index("s")
        w = s * _NC + c
        one16 = jnp.full((16,), 1.0, jnp.float32)
        zero16 = jnp.zeros((16,), jnp.float32)

        def fill(r, carry):
            ones_v[r, :] = one16
            stage_v[r, :] = zero16
            return carry
        lax.fori_loop(0, _CHUNK, fill, 0)

        base = s * rows_pt
        for i in range(rows_pt // _CHUNK):
            pltpu.sync_copy(stage_v, acc.at[pl.ds(base + i * _CHUNK, _CHUNK)])
        plsc.subcore_barrier()

        pltpu.sync_copy(col_h.at[w], cidx)

        def body(j, carry):
            pltpu.sync_copy(ones_v, acc.at[cidx.at[j]], add=True)
            return carry
        lax.fori_loop(0, K, body, 0)
        plsc.subcore_barrier()

        for i in range(rows_pt // _CHUNK):
            sl = pl.ds(base + i * _CHUNK, _CHUNK)
            pltpu.sync_copy(acc.at[sl], stage_v)
            pltpu.sync_copy(stage_v, out_h.at[c, sl])

    return deg_kernel(col_r)


def _sc_aggregate(table, row_r, col_r, n_pad):
    K = row_r.shape[1]
    Hk = table.shape[1]
    rows_pt = n_pad // _NS

    @functools.partial(
        pl.kernel,
        out_type=jax.ShapeDtypeStruct((_NC, n_pad, Hk), jnp.float32),
        mesh=_vsc_mesh(),
        compiler_params=pltpu.CompilerParams(use_tc_tiling_on_sc=False),
        scratch_types=[
            pltpu.VMEM((K, _CHUNK), jnp.int32),
            pltpu.VMEM((K, _CHUNK), jnp.int32),
            pltpu.VMEM((_CHUNK, Hk), jnp.float32),
            pltpu.VMEM((_CHUNK, Hk), jnp.float32),
            pltpu.VMEM_SHARED((n_pad, Hk), jnp.float32),
            pltpu.VMEM_SHARED((n_pad, Hk), jnp.float32),
            pltpu.SemaphoreType.DMA,
            pltpu.SemaphoreType.DMA,
        ],
    )
    def agg_kernel(table_h, row_h, col_h, out_h, ridx, cidx, gbuf, gb1, tab,
                   acc, sem0, sem1):
        c = lax.axis_index("c")
        s = lax.axis_index("s")
        w = s * _NC + c
        zero16 = jnp.zeros((16,), jnp.float32)

        def fill(r, carry):
            for jj in range(Hk // 16):
                gbuf[r, pl.ds(jj * 16, 16)] = zero16
            return carry
        lax.fori_loop(0, _CHUNK, fill, 0)

        base = s * rows_pt
        sl_mine = pl.ds(base, rows_pt)
        for i in range(rows_pt // _CHUNK):
            pltpu.sync_copy(gbuf, acc.at[pl.ds(base + i * _CHUNK, _CHUNK)])
        pltpu.sync_copy(table_h.at[sl_mine], tab.at[sl_mine])
        plsc.subcore_barrier()

        pltpu.sync_copy(row_h.at[w], ridx)
        pltpu.sync_copy(col_h.at[w], cidx)

        pltpu.async_copy(tab.at[ridx.at[0]], gbuf, sem0)

        def body(j2, carry):
            j = 2 * j2
            pltpu.async_copy(tab.at[ridx.at[j + 1]], gb1, sem1)
            pltpu.make_async_copy(tab.at[ridx.at[j]], gbuf, sem0).wait()
            pltpu.sync_copy(gbuf, acc.at[cidx.at[j]], add=True)

            @pl.when(j + 2 < K)
            def _():
                pltpu.async_copy(tab.at[ridx.at[j + 2]], gbuf, sem0)
            pltpu.make_async_copy(tab.at[ridx.at[j + 1]], gb1, sem1).wait()
            pltpu.sync_copy(gb1, acc.at[cidx.at[j + 1]], add=True)
            return carry
        lax.fori_loop(0, K // 2, body, 0)
        plsc.subcore_barrier()

        for i in range(rows_pt // _CHUNK):
            sl = pl.ds(base + i * _CHUNK, _CHUNK)
            pltpu.sync_copy(acc.at[sl], gbuf)
            pltpu.sync_copy(gbuf, out_h.at[c, sl])

    return agg_kernel(table, row_r, col_r)


def _tc_matmul(x, W, br):
    n, d = x.shape
    h = W.shape[1]

    def body(xr, wr, orf):
        orf[...] = jnp.dot(xr[...], wr[...], preferred_element_type=jnp.float32)

    return pl.pallas_call(
        body,
        grid=(n // br,),
        in_specs=[pl.BlockSpec((br, d), lambda i: (i, 0)),
                  pl.BlockSpec((d, h), lambda i: (0, 0))],
        out_specs=pl.BlockSpec((br, h), lambda i: (i, 0)),
        out_shape=jax.ShapeDtypeStruct((n, h), jnp.float32),
    )(x, W)


def _tc_scale(deg2, h1, br, n_pad):
    n, h = h1.shape

    def body(dr, hr, hs_ref, dv_ref):
        d = dr[...]
        deg = d[0, :, 0:1] + d[1, :, 0:1] + 1.0
        dinv = lax.rsqrt(deg)
        hs_ref[...] = dinv * hr[...]
        dv_ref[...] = jnp.broadcast_to(dinv, (br, 8))

    return pl.pallas_call(
        body,
        grid=(n // br,),
        in_specs=[pl.BlockSpec((2, br, 16), lambda i: (0, i, 0)),
                  pl.BlockSpec((br, h), lambda i: (i, 0))],
        out_specs=[pl.BlockSpec((br, h), lambda i: (i, 0)),
                   pl.BlockSpec((br, 8), lambda i: (i, 0))],
        out_shape=[jax.ShapeDtypeStruct((n_pad, h), jnp.float32),
                   jax.ShapeDtypeStruct((n, 8), jnp.float32)],
    )(deg2, h1)


def _tc_fuse(agg, hs, dinv8, b, g, be, W, br, rescale, out_bias=None,
             out_rows=None):
    n = dinv8.shape[0]
    h = hs.shape[1]
    h_out = W.shape[1]
    if out_rows is None:
        out_rows = n
    has_ob = out_bias is not None

    def body(*refs):
        if has_ob:
            ar, hr, dvr, br_, gr, ber, wr, obr, orf = refs
        else:
            ar, hr, dvr, br_, gr, ber, wr, orf = refs
        a = ar[...]
        dv = dvr[...][:, 0:1]
        t = dv * (a[0] + a[1] + hr[...]) + br_[...]
        t = jnp.maximum((_BNC * gr[...]) * t + ber[...], 0.0)
        o = jnp.dot(t, wr[...], preferred_element_type=jnp.float32)
        if has_ob:
            o = o + obr[...]
        if rescale:
            o = dv * o
        orf[...] = o

    in_specs = [pl.BlockSpec((2, br, h), lambda i: (0, i, 0)),
                pl.BlockSpec((br, h), lambda i: (i, 0)),
                pl.BlockSpec((br, 8), lambda i: (i, 0)),
                pl.BlockSpec((h,), lambda i: (0,)),
                pl.BlockSpec((h,), lambda i: (0,)),
                pl.BlockSpec((h,), lambda i: (0,)),
                pl.BlockSpec((h, h_out), lambda i: (0, 0))]
    args = [agg, hs, dinv8, b, g, be, W]
    if has_ob:
        in_specs.append(pl.BlockSpec((h_out,), lambda i: (0,)))
        args.append(out_bias)
    return pl.pallas_call(
        body,
        grid=(n // br,),
        in_specs=in_specs,
        out_specs=pl.BlockSpec((br, h_out), lambda i: (i, 0)),
        out_shape=jax.ShapeDtypeStruct((out_rows, h_out), jnp.float32),
    )(*args)


def kernel(x, edge_index, W1, b1, g1, be1, W2, b2, g2, be2, Wl, bl):
    n, d = x.shape
    e = edge_index.shape[1]
    out_w = Wl.shape[1]

    k_chunks = -(-e // (_NW * _CHUNK))
    if k_chunks % 2:
        k_chunks += 1
    e_pad = _NW * k_chunks * _CHUNK
    n_pad = -(-(n + 1) // (_NS * _CHUNK)) * (_NS * _CHUNK)

    row = edge_index[0]
    col = edge_index[1]
    row_r = jnp.concatenate(
        [row, jnp.zeros((e_pad - e,), jnp.int32)]).reshape(_NW, k_chunks, _CHUNK)
    col_r = jnp.concatenate(
        [col, jnp.full((e_pad - e,), n, jnp.int32)]).reshape(_NW, k_chunks, _CHUNK)

    br = 1000 if n % 1000 == 0 else 8 * (n // 8)

    deg2 = _sc_degree(col_r, n_pad)
    h1 = _tc_matmul(x, W1, br)
    hs1, dinv8 = _tc_scale(deg2, h1, br, n_pad)

    agg1 = _sc_aggregate(hs1, row_r, col_r, n_pad)
    hs2 = _tc_fuse(agg1, hs1, dinv8, b1, g1, be1, W2, br, rescale=True,
                   out_rows=n_pad)

    agg2 = _sc_aggregate(hs2, row_r, col_r, n_pad)
    wl_pad = jnp.zeros((Wl.shape[0], 128), jnp.float32).at[:, :out_w].set(Wl)
    bl_pad = jnp.zeros((128,), jnp.float32).at[:out_w].set(bl)
    out = _tc_fuse(agg2, hs2, dinv8, b2, g2, be2, wl_pad, br, rescale=False,
                   out_bias=bl_pad)
    return out[:, :out_w]


# speedup vs baseline: 2.4374x; 1.0641x over previous
import functools

import jax
import jax.numpy as jnp
from jax import lax
from jax.experimental import pallas as pl
from jax.experimental.pallas import tpu as pltpu
from jax.experimental.pallas import tpu_sc as plsc

_NC = 2
_NS = 16
_NW = _NC * _NS
_CHUNK = 128
_BNC = float(1.0 / (1.0 + 1e-5) ** 0.5)


def _vsc_mesh():
    return plsc.VectorSubcoreMesh(
        core_axis_name="c", subcore_axis_name="s",
        num_cores=_NC, num_subcores=_NS)


def _copy_chunks(src2d, dst, w, base, extra, n_chunks):
    pltpu.sync_copy(src2d.at[pl.ds(w * base, base)], dst.at[pl.ds(0, base)])
    if extra:
        @pl.when(w < extra)
        def _():
            pltpu.sync_copy(src2d.at[n_chunks - extra + w], dst.at[base])


def _sc_degree(ei3, n_pad, base, extra):
    n_chunks = ei3.shape[1]
    rows_pt = n_pad // _NS

    @functools.partial(
        pl.kernel,
        out_type=jax.ShapeDtypeStruct((_NC, n_pad, 16), jnp.float32),
        mesh=_vsc_mesh(),
        compiler_params=pltpu.CompilerParams(use_tc_tiling_on_sc=False),
        scratch_types=[
            pltpu.VMEM((base + 1, _CHUNK), jnp.int32),
            pltpu.VMEM((_CHUNK, 16), jnp.float32),
            pltpu.VMEM((_CHUNK, 16), jnp.float32),
            pltpu.VMEM_SHARED((n_pad, 16), jnp.float32),
        ],
    )
    def deg_kernel(ei_h, out_h, cidx, ones_v, stage_v, acc):
        c = lax.axis_index("c")
        s = lax.axis_index("s")
        w = s * _NC + c
        one16 = jnp.full((16,), 1.0, jnp.float32)
        zero16 = jnp.zeros((16,), jnp.float32)

        def fill(r, carry):
            ones_v[r, :] = one16
            stage_v[r, :] = zero16
            return carry
        lax.fori_loop(0, _CHUNK, fill, 0)

        base_row = s * rows_pt
        for i in range(rows_pt // _CHUNK):
            pltpu.sync_copy(stage_v, acc.at[pl.ds(base_row + i * _CHUNK, _CHUNK)])
        plsc.subcore_barrier()

        _copy_chunks(ei_h.at[1], cidx, w, base, extra, n_chunks)
        my_k = base + jnp.where(w < extra, 1, 0) if extra else base

        def body(j, carry):
            pltpu.sync_copy(ones_v, acc.at[cidx.at[j]], add=True)
            return carry
        lax.fori_loop(0, my_k, body, 0)
        plsc.subcore_barrier()

        for i in range(rows_pt // _CHUNK):
            sl = pl.ds(base_row + i * _CHUNK, _CHUNK)
            pltpu.sync_copy(acc.at[sl], stage_v)
            pltpu.sync_copy(stage_v, out_h.at[c, sl])

    return deg_kernel(ei3)


def _sc_aggregate(table, ei3, n_pad, base, extra):
    n_chunks = ei3.shape[1]
    Hk = table.shape[1]
    rows_pt = n_pad // _NS
    pairs = base // 2

    @functools.partial(
        pl.kernel,
        out_type=jax.ShapeDtypeStruct((_NC, n_pad, Hk), jnp.float32),
        mesh=_vsc_mesh(),
        compiler_params=pltpu.CompilerParams(use_tc_tiling_on_sc=False),
        scratch_types=[
            pltpu.VMEM((base + 1, _CHUNK), jnp.int32),
            pltpu.VMEM((base + 1, _CHUNK), jnp.int32),
            pltpu.VMEM((_CHUNK, Hk), jnp.float32),
            pltpu.VMEM((_CHUNK, Hk), jnp.float32),
            pltpu.VMEM_SHARED((n_pad, Hk), jnp.float32),
            pltpu.VMEM_SHARED((n_pad, Hk), jnp.float32),
            pltpu.SemaphoreType.DMA,
            pltpu.SemaphoreType.DMA,
        ],
    )
    def agg_kernel(table_h, ei_h, out_h, ridx, cidx, gbuf, gb1, tab,
                   acc, sem0, sem1):
        c = lax.axis_index("c")
        s = lax.axis_index("s")
        w = s * _NC + c
        zero16 = jnp.zeros((16,), jnp.float32)

        def fill(r, carry):
            for jj in range(Hk // 16):
                gbuf[r, pl.ds(jj * 16, 16)] = zero16
            return carry
        lax.fori_loop(0, _CHUNK, fill, 0)

        base_row = s * rows_pt
        sl_mine = pl.ds(base_row, rows_pt)
        for i in range(rows_pt // _CHUNK):
            pltpu.sync_copy(gbuf, acc.at[pl.ds(base_row + i * _CHUNK, _CHUNK)])
        pltpu.sync_copy(table_h.at[sl_mine], tab.at[sl_mine])
        plsc.subcore_barrier()

        _copy_chunks(ei_h.at[0], ridx, w, base, extra, n_chunks)
        _copy_chunks(ei_h.at[1], cidx, w, base, extra, n_chunks)

        pltpu.async_copy(tab.at[ridx.at[0]], gbuf, sem0)

        def body(j2, carry):
            j = 2 * j2
            pltpu.async_copy(tab.at[ridx.at[j + 1]], gb1, sem1)
            pltpu.make_async_copy(tab.at[ridx.at[j]], gbuf, sem0).wait()
            pltpu.sync_copy(gbuf, acc.at[cidx.at[j]], add=True)

            @pl.when(j + 2 < 2 * pairs)
            def _():
                pltpu.async_copy(tab.at[ridx.at[j + 2]], gbuf, sem0)
            pltpu.make_async_copy(tab.at[ridx.at[j + 1]], gb1, sem1).wait()
            pltpu.sync_copy(gb1, acc.at[cidx.at[j + 1]], add=True)
            return carry
        lax.fori_loop(0, pairs, body, 0)

        tail = list(range(2 * pairs, base)) + ([base] if extra else [])
        for j in tail:
            cond = (w >= 0) if j < base else (w < extra)

            @pl.when(cond)
            def _(j=j):
                pltpu.sync_copy(tab.at[ridx.at[j]], gbuf)
                pltpu.sync_copy(gbuf, acc.at[cidx.at[j]], add=True)
        plsc.subcore_barrier()

        for i in range(rows_pt // _CHUNK):
            sl = pl.ds(base_row + i * _CHUNK, _CHUNK)
            pltpu.sync_copy(acc.at[sl], gbuf)
            pltpu.sync_copy(gbuf, out_h.at[c, sl])

    return agg_kernel(table, ei3)


def _tc_matmul(x, W, br):
    n, d = x.shape
    h = W.shape[1]

    def body(xr, wr, orf):
        orf[...] = jnp.dot(xr[...], wr[...], preferred_element_type=jnp.float32)

    return pl.pallas_call(
        body,
        grid=(n // br,),
        in_specs=[pl.BlockSpec((br, d), lambda i: (i, 0)),
                  pl.BlockSpec((d, h), lambda i: (0, 0))],
        out_specs=pl.BlockSpec((br, h), lambda i: (i, 0)),
        out_shape=jax.ShapeDtypeStruct((n, h), jnp.float32),
    )(x, W)


def _tc_scale(deg2, h1, br, n_pad):
    n, h = h1.shape

    def body(dr, hr, hs_ref, dv_ref):
        d = dr[...]
        deg = d[0, :, 0:1] + d[1, :, 0:1] + 1.0
        dinv = lax.rsqrt(deg)
        hs_ref[...] = dinv * hr[...]
        dv_ref[...] = jnp.broadcast_to(dinv, (br, 8))

    return pl.pallas_call(
        body,
        grid=(n // br,),
        in_specs=[pl.BlockSpec((2, br, 16), lambda i: (0, i, 0)),
                  pl.BlockSpec((br, h), lambda i: (i, 0))],
        out_specs=[pl.BlockSpec((br, h), lambda i: (i, 0)),
                   pl.BlockSpec((br, 8), lambda i: (i, 0))],
        out_shape=[jax.ShapeDtypeStruct((n_pad, h), jnp.float32),
                   jax.ShapeDtypeStruct((n, 8), jnp.float32)],
    )(deg2, h1)


def _tc_fuse(agg, hs, dinv8, b, g, be, W, br, rescale, out_bias=None,
             out_rows=None):
    n = dinv8.shape[0]
    h = hs.shape[1]
    h_out = W.shape[1]
    if out_rows is None:
        out_rows = n
    has_ob = out_bias is not None

    def body(*refs):
        if has_ob:
            ar, hr, dvr, br_, gr, ber, wr, obr, orf = refs
        else:
            ar, hr, dvr, br_, gr, ber, wr, orf = refs
        a = ar[...]
        dv = dvr[...][:, 0:1]
        t = dv * (a[0] + a[1] + hr[...]) + br_[...]
        t = jnp.maximum((_BNC * gr[...]) * t + ber[...], 0.0)
        o = jnp.dot(t, wr[...], preferred_element_type=jnp.float32)
        if has_ob:
            o = o + obr[...]
        if rescale:
            o = dv * o
        orf[...] = o

    in_specs = [pl.BlockSpec((2, br, h), lambda i: (0, i, 0)),
                pl.BlockSpec((br, h), lambda i: (i, 0)),
                pl.BlockSpec((br, 8), lambda i: (i, 0)),
                pl.BlockSpec((h,), lambda i: (0,)),
                pl.BlockSpec((h,), lambda i: (0,)),
                pl.BlockSpec((h,), lambda i: (0,)),
                pl.BlockSpec((h, h_out), lambda i: (0, 0))]
    args = [agg, hs, dinv8, b, g, be, W]
    if has_ob:
        in_specs.append(pl.BlockSpec((h_out,), lambda i: (0,)))
        args.append(out_bias)
    return pl.pallas_call(
        body,
        grid=(n // br,),
        in_specs=in_specs,
        out_specs=pl.BlockSpec((br, h_out), lambda i: (i, 0)),
        out_shape=jax.ShapeDtypeStruct((out_rows, h_out), jnp.float32),
    )(*args)


def kernel(x, edge_index, W1, b1, g1, be1, W2, b2, g2, be2, Wl, bl):
    n, d = x.shape
    e = edge_index.shape[1]
    out_w = Wl.shape[1]

    if e % _CHUNK:
        pad = _CHUNK - e % _CHUNK
        edge_index = jnp.concatenate(
            [edge_index,
             jnp.concatenate([jnp.zeros((1, pad), jnp.int32),
                              jnp.full((1, pad), n, jnp.int32)])], axis=1)
    n_chunks = edge_index.shape[1] // _CHUNK
    ei3 = edge_index.reshape(2, n_chunks, _CHUNK)
    cbase = n_chunks // _NW
    cextra = n_chunks % _NW
    n_pad = -(-(n + 1) // (_NS * _CHUNK)) * (_NS * _CHUNK)

    br = 1000 if n % 1000 == 0 else 8 * (n // 8)

    deg2 = _sc_degree(ei3, n_pad, cbase, cextra)
    h1 = _tc_matmul(x, W1, br)
    hs1, dinv8 = _tc_scale(deg2, h1, br, n_pad)

    agg1 = _sc_aggregate(hs1, ei3, n_pad, cbase, cextra)
    hs2 = _tc_fuse(agg1, hs1, dinv8, b1, g1, be1, W2, br, rescale=True,
                   out_rows=n_pad)

    agg2 = _sc_aggregate(hs2, ei3, n_pad, cbase, cextra)
    wl_pad = jnp.zeros((Wl.shape[0], 128), jnp.float32).at[:, :out_w].set(Wl)
    bl_pad = jnp.zeros((128,), jnp.float32).at[:out_w].set(bl)
    out = _tc_fuse(agg2, hs2, dinv8, b2, g2, be2, wl_pad, br, rescale=False,
                   out_bias=bl_pad)
    return out[:, :out_w]
